# baseline (speedup 1.0000x reference)
import functools

import jax
import jax.numpy as jnp
from jax import lax
from jax.experimental import pallas as pl
from jax.experimental.pallas import tpu as pltpu
from jax.experimental.pallas import tpu_sc as plsc

N_NODES = 10000
D = 128
N_EDGES = 320000

NC = 2
NS = 16
NT = NC * NS

CB = 64
NBUF = 4
KITER = -(-N_EDGES // (NT * CB * NBUF))
NCHUNK_SC = NBUF * KITER
NCHUNK_IO = NCHUNK_SC + NBUF
E_PAD = NT * CB * NCHUNK_IO
NP = 10112
RPT = NP // NS

_STRIPE_CHUNKS = [CB] * (RPT // CB) + ([RPT % CB] if RPT % CB else [])


def _sc_body(x_hbm, src_hbm, dst_hbm, sum_out, cnt_out, acc, *refs):
    src_v = refs[0:NBUF]
    dst_v = refs[NBUF:2 * NBUF]
    rows = refs[2 * NBUF:3 * NBUF]
    s_is = refs[3 * NBUF:4 * NBUF]
    s_id = refs[4 * NBUF:5 * NBUF]
    s_g = refs[5 * NBUF:6 * NBUF]

    cid = lax.axis_index("c")
    sid = lax.axis_index("s")
    tid = cid * NS + sid
    base = sid * RPT
    out_base = cid * NP + base

    def fill_rows(buf, val16):
        def fill(i, carry):
            for j in range(D // 16):
                buf[i, pl.ds(j * 16, 16)] = val16
            return carry
        lax.fori_loop(0, CB, fill, 0)

    def zero_acc():
        off = 0
        for sz in _STRIPE_CHUNKS:
            pltpu.sync_copy(rows[0].at[pl.ds(0, sz)],
                            acc.at[pl.ds(base + off, sz)])
            off += sz

    def writeout(dest):
        off = 0
        for sz in _STRIPE_CHUNKS:
            pltpu.sync_copy(acc.at[pl.ds(base + off, sz)],
                            rows[0].at[pl.ds(0, sz)])
            pltpu.sync_copy(rows[0].at[pl.ds(0, sz)],
                            dest.at[pl.ds(out_base + off, sz)])
            off += sz

    def wait_idx(sem, buf):
        pltpu.make_async_copy(src_hbm.at[tid, 0], buf.at[0], sem).wait()

    def wait_gather(sem, src_idx, buf):
        pltpu.make_async_copy(x_hbm.at[src_idx.at[0]], buf, sem).wait()

    def prefetch_idx(g, b):
        pltpu.async_copy(src_hbm.at[tid, g], src_v[b].at[0], s_is[b])
        pltpu.async_copy(dst_hbm.at[tid, g], dst_v[b].at[0], s_id[b])

    def start_gather(b):
        pltpu.async_copy(x_hbm.at[src_v[b].at[0]], rows[b], s_g[b])

    zeros16 = jnp.zeros((16,), jnp.float32)
    ones16 = jnp.ones((16,), jnp.float32)

    fill_rows(rows[0], zeros16)
    zero_acc()
    plsc.subcore_barrier()

    for b in range(NBUF):
        prefetch_idx(b, b)
    for b in range(NBUF - 1):
        wait_idx(s_is[b], src_v[b])
        start_gather(b)

    def pbody(i, carry):
        g0 = NBUF * i
        for b in range(NBUF):
            b3 = (b + NBUF - 1) % NBUF
            wait_gather(s_g[b], src_v[b], rows[b])
            wait_idx(s_id[b], dst_v[b])
            pltpu.sync_copy(rows[b], acc.at[dst_v[b].at[0]], add=True)
            prefetch_idx(g0 + b + NBUF, b)
            wait_idx(s_is[b3], src_v[b3])
            start_gather(b3)
        return carry

    lax.fori_loop(0, KITER, pbody, 0)
    for b in range(NBUF - 1):
        wait_gather(s_g[b], src_v[b], rows[b])
    wait_idx(s_is[NBUF - 1], src_v[NBUF - 1])
    for b in range(NBUF):
        wait_idx(s_id[b], dst_v[b])
    plsc.subcore_barrier()
    writeout(sum_out)

    fill_rows(rows[0], zeros16)
    zero_acc()
    fill_rows(rows[1], ones16)
    plsc.subcore_barrier()

    for b in range(NBUF):
        pltpu.async_copy(dst_hbm.at[tid, b], dst_v[b].at[0], s_id[b])

    def cbody(i, carry):
        g0 = NBUF * i
        for b in range(NBUF):
            wait_idx(s_id[b], dst_v[b])
            pltpu.sync_copy(rows[1], acc.at[dst_v[b].at[0]], add=True)
            pltpu.async_copy(dst_hbm.at[tid, g0 + b + NBUF],
                             dst_v[b].at[0], s_id[b])
        return carry

    lax.fori_loop(0, KITER, cbody, 0)
    for b in range(NBUF):
        wait_idx(s_id[b], dst_v[b])
    plsc.subcore_barrier()
    writeout(cnt_out)


_sc_aggregate = functools.partial(
    pl.kernel,
    out_type=(jax.ShapeDtypeStruct((NC * NP, D), jnp.float32),
              jax.ShapeDtypeStruct((NC * NP, D), jnp.float32)),
    mesh=plsc.VectorSubcoreMesh(core_axis_name="c", subcore_axis_name="s",
                                num_cores=NC, num_subcores=NS),
    scratch_types=(
        [pltpu.VMEM_SHARED((NP, D), jnp.float32)]
        + [pltpu.VMEM((1, CB), jnp.int32)] * (2 * NBUF)
        + [pltpu.VMEM((CB, D), jnp.float32)] * NBUF
        + [pltpu.SemaphoreType.DMA] * (3 * NBUF)
    ),
)(_sc_body)


ROWS_BLK = 1000


def _tc_body(x_ref, s_ref, c_ref, w_ref, b_ref, o_ref):
    s = s_ref[0] + s_ref[1]
    c = c_ref[0, :, 0:1] + c_ref[1, :, 0:1]
    agg = s / jnp.maximum(c, 1.0)
    t = lax.dot_general(agg, w_ref[...], (((1,), (1,)), ((), ())),
                        preferred_element_type=jnp.float32)
    o_ref[...] = x_ref[...] + t + b_ref[...]


_tc_combine = pl.pallas_call(
    _tc_body,
    grid=(N_NODES // ROWS_BLK,),
    in_specs=[
        pl.BlockSpec((ROWS_BLK, D), lambda g: (g, 0)),
        pl.BlockSpec((NC, ROWS_BLK, D), lambda g: (0, g, 0)),
        pl.BlockSpec((NC, ROWS_BLK, D), lambda g: (0, g, 0)),
        pl.BlockSpec((D, D), lambda g: (0, 0)),
        pl.BlockSpec((1, D), lambda g: (0, 0)),
    ],
    out_specs=pl.BlockSpec((ROWS_BLK, D), lambda g: (g, 0)),
    out_shape=jax.ShapeDtypeStruct((N_NODES, D), jnp.float32),
)


def kernel(x, edge_index, W, b):
    src = edge_index[0].astype(jnp.int32)
    dst = edge_index[1].astype(jnp.int32)
    pad = NT * NCHUNK_SC * CB - N_EDGES
    src = jnp.concatenate([src, jnp.zeros((pad,), jnp.int32)])
    dst = jnp.concatenate([dst, jnp.full((pad,), N_NODES, jnp.int32)])
    src3 = jnp.concatenate(
        [src.reshape(NT, NCHUNK_SC, CB),
         jnp.zeros((NT, NBUF, CB), jnp.int32)], axis=1)
    dst3 = jnp.concatenate(
        [dst.reshape(NT, NCHUNK_SC, CB),
         jnp.full((NT, NBUF, CB), N_NODES, jnp.int32)], axis=1)
    sums, cnts = _sc_aggregate(x, src3, dst3)
    sums = sums.reshape(NC, NP, D)
    cnts = cnts.reshape(NC, NP, D)
    return _tc_combine(x, sums, cnts, W, b.reshape(1, D))


# speedup vs baseline: 1.0952x; 1.0952x over previous
import functools

import jax
import jax.numpy as jnp
from jax import lax
from jax.experimental import pallas as pl
from jax.experimental.pallas import tpu as pltpu
from jax.experimental.pallas import tpu_sc as plsc

N_NODES = 10000
D = 128
N_EDGES = 320000

NC = 2
NS = 16
NT = NC * NS

CB = 128
NITER = -(-N_EDGES // (NT * CB * 2))
NCHUNK_SC = 2 * NITER
NCHUNK_IO = NCHUNK_SC + 2
E_PAD = NT * CB * NCHUNK_IO
NP = 10112
RPT = NP // NS

_STRIPE_CHUNKS = [CB] * (RPT // CB) + ([RPT % CB] if RPT % CB else [])


def _sc_body(x_hbm, src_hbm, dst_hbm, sum_out, cnt_out,
             acc, src_v0, src_v1, dst_v0, dst_v1, rows_a, rows_b,
             s_is0, s_is1, s_id0, s_id1, s_ga, s_gb):
    cid = lax.axis_index("c")
    sid = lax.axis_index("s")
    tid = cid * NS + sid
    base = sid * RPT
    out_base = cid * NP + base

    def fill_rows(buf, val16):
        def fill(i, carry):
            for j in range(D // 16):
                buf[i, pl.ds(j * 16, 16)] = val16
            return carry
        lax.fori_loop(0, CB, fill, 0)

    def zero_acc():
        off = 0
        for sz in _STRIPE_CHUNKS:
            pltpu.sync_copy(rows_a.at[pl.ds(0, sz)],
                            acc.at[pl.ds(base + off, sz)])
            off += sz

    def writeout(dest):
        off = 0
        for sz in _STRIPE_CHUNKS:
            pltpu.sync_copy(acc.at[pl.ds(base + off, sz)],
                            rows_a.at[pl.ds(0, sz)])
            pltpu.sync_copy(rows_a.at[pl.ds(0, sz)],
                            dest.at[pl.ds(out_base + off, sz)])
            off += sz

    def wait_idx(sem, buf):
        pltpu.make_async_copy(src_hbm.at[tid, 0], buf.at[0], sem).wait()

    def wait_gather(sem, src_idx, buf):
        pltpu.make_async_copy(x_hbm.at[src_idx.at[0]], buf, sem).wait()

    zeros16 = jnp.zeros((16,), jnp.float32)
    ones16 = jnp.ones((16,), jnp.float32)

    with jax.named_scope("sc_zero1"):
        fill_rows(rows_a, zeros16)
        zero_acc()
        plsc.subcore_barrier()

    pltpu.async_copy(src_hbm.at[tid, 0], src_v0.at[0], s_is0)
    pltpu.async_copy(dst_hbm.at[tid, 0], dst_v0.at[0], s_id0)
    pltpu.async_copy(src_hbm.at[tid, 1], src_v1.at[0], s_is1)
    pltpu.async_copy(dst_hbm.at[tid, 1], dst_v1.at[0], s_id1)
    wait_idx(s_is0, src_v0)
    pltpu.async_copy(x_hbm.at[src_v0.at[0]], rows_a, s_ga)

    def pbody(i, carry):
        g2 = 2 * i + 2
        g3 = 2 * i + 3
        wait_idx(s_is1, src_v1)
        wait_gather(s_ga, src_v0, rows_a)
        pltpu.async_copy(x_hbm.at[src_v1.at[0]], rows_b, s_gb)
        wait_idx(s_id0, dst_v0)
        pltpu.sync_copy(rows_a, acc.at[dst_v0.at[0]], add=True)
        pltpu.async_copy(src_hbm.at[tid, g2], src_v0.at[0], s_is0)
        pltpu.async_copy(dst_hbm.at[tid, g2], dst_v0.at[0], s_id0)
        wait_idx(s_is0, src_v0)
        wait_gather(s_gb, src_v1, rows_b)
        pltpu.async_copy(x_hbm.at[src_v0.at[0]], rows_a, s_ga)
        wait_idx(s_id1, dst_v1)
        pltpu.sync_copy(rows_b, acc.at[dst_v1.at[0]], add=True)
        pltpu.async_copy(src_hbm.at[tid, g3], src_v1.at[0], s_is1)
        pltpu.async_copy(dst_hbm.at[tid, g3], dst_v1.at[0], s_id1)
        return carry

    with jax.named_scope("sc_p1"):
        lax.fori_loop(0, NITER, pbody, 0)
        wait_gather(s_ga, src_v0, rows_a)
        wait_idx(s_id0, dst_v0)
        wait_idx(s_is1, src_v1)
        wait_idx(s_id1, dst_v1)
        plsc.subcore_barrier()
    with jax.named_scope("sc_wo1"):
        writeout(sum_out)

    with jax.named_scope("sc_zero2"):
        fill_rows(rows_a, zeros16)
        zero_acc()
        fill_rows(rows_b, ones16)
        plsc.subcore_barrier()

    pltpu.async_copy(dst_hbm.at[tid, 0], dst_v0.at[0], s_id0)
    pltpu.async_copy(dst_hbm.at[tid, 1], dst_v1.at[0], s_id1)

    def cbody(i, carry):
        g2 = 2 * i + 2
        g3 = 2 * i + 3
        wait_idx(s_id0, dst_v0)
        pltpu.sync_copy(rows_b, acc.at[dst_v0.at[0]], add=True)
        pltpu.async_copy(dst_hbm.at[tid, g2], dst_v0.at[0], s_id0)
        wait_idx(s_id1, dst_v1)
        pltpu.sync_copy(rows_b, acc.at[dst_v1.at[0]], add=True)
        pltpu.async_copy(dst_hbm.at[tid, g3], dst_v1.at[0], s_id1)
        return carry

    with jax.named_scope("sc_p2"):
        lax.fori_loop(0, NITER, cbody, 0)
        wait_idx(s_id0, dst_v0)
        wait_idx(s_id1, dst_v1)
        plsc.subcore_barrier()
    with jax.named_scope("sc_wo2"):
        writeout(cnt_out)


_sc_aggregate = functools.partial(
    pl.kernel,
    out_type=(jax.ShapeDtypeStruct((NC * NP, D), jnp.float32),
              jax.ShapeDtypeStruct((NC * NP, D), jnp.float32)),
    mesh=plsc.VectorSubcoreMesh(core_axis_name="c", subcore_axis_name="s",
                                num_cores=NC, num_subcores=NS),
    scratch_types=[
        pltpu.VMEM_SHARED((NP, D), jnp.float32),
        pltpu.VMEM((1, CB), jnp.int32),
        pltpu.VMEM((1, CB), jnp.int32),
        pltpu.VMEM((1, CB), jnp.int32),
        pltpu.VMEM((1, CB), jnp.int32),
        pltpu.VMEM((CB, D), jnp.float32),
        pltpu.VMEM((CB, D), jnp.float32),
        pltpu.SemaphoreType.DMA,
        pltpu.SemaphoreType.DMA,
        pltpu.SemaphoreType.DMA,
        pltpu.SemaphoreType.DMA,
        pltpu.SemaphoreType.DMA,
        pltpu.SemaphoreType.DMA,
    ],
)(_sc_body)


ROWS_BLK = 1000


def _tc_body(x_ref, s_ref, c_ref, w_ref, b_ref, o_ref):
    s = s_ref[0] + s_ref[1]
    c = c_ref[0, :, 0:1] + c_ref[1, :, 0:1]
    agg = s / jnp.maximum(c, 1.0)
    t = lax.dot_general(agg, w_ref[...], (((1,), (1,)), ((), ())),
                        preferred_element_type=jnp.float32)
    o_ref[...] = x_ref[...] + t + b_ref[...]


_tc_combine = pl.pallas_call(
    _tc_body,
    grid=(N_NODES // ROWS_BLK,),
    in_specs=[
        pl.BlockSpec((ROWS_BLK, D), lambda g: (g, 0)),
        pl.BlockSpec((NC, ROWS_BLK, D), lambda g: (0, g, 0)),
        pl.BlockSpec((NC, ROWS_BLK, D), lambda g: (0, g, 0)),
        pl.BlockSpec((D, D), lambda g: (0, 0)),
        pl.BlockSpec((1, D), lambda g: (0, 0)),
    ],
    out_specs=pl.BlockSpec((ROWS_BLK, D), lambda g: (g, 0)),
    out_shape=jax.ShapeDtypeStruct((N_NODES, D), jnp.float32),
)


def kernel(x, edge_index, W, b):
    src = edge_index[0].astype(jnp.int32)
    dst = edge_index[1].astype(jnp.int32)
    pad = NT * NCHUNK_SC * CB - N_EDGES
    src = jnp.concatenate([src, jnp.zeros((pad,), jnp.int32)])
    dst = jnp.concatenate([dst, jnp.full((pad,), N_NODES, jnp.int32)])
    src3 = jnp.concatenate(
        [src.reshape(NT, NCHUNK_SC, CB),
         jnp.zeros((NT, 2, CB), jnp.int32)], axis=1)
    dst3 = jnp.concatenate(
        [dst.reshape(NT, NCHUNK_SC, CB),
         jnp.full((NT, 2, CB), N_NODES, jnp.int32)], axis=1)
    sums, cnts = _sc_aggregate(x, src3, dst3)
    sums = sums.reshape(NC, NP, D)
    cnts = cnts.reshape(NC, NP, D)
    return _tc_combine(x, sums, cnts, W, b.reshape(1, D))


# speedup vs baseline: 1.6148x; 1.4744x over previous
import functools

import jax
import jax.numpy as jnp
from jax import lax
from jax.experimental import pallas as pl
from jax.experimental.pallas import tpu as pltpu
from jax.experimental.pallas import tpu_sc as plsc

N_NODES = 10000
D = 128
N_EDGES = 320000

NC = 2
NS = 16
NT = NC * NS

CB = 32
NITER = -(-N_EDGES // (NT * CB * 2))
NCHUNK_SC = 2 * NITER
NCHUNK_IO = NCHUNK_SC + 2
E_PAD = NT * CB * NCHUNK_IO
NP = 10112
RPT = NP // NS

_STRIPE_CHUNKS = [CB] * (RPT // CB) + ([RPT % CB] if RPT % CB else [])


def _sc_body(x_hbm, src_hbm, dst_hbm, sum_out, cnt_out,
             acc, src_v0, src_v1, dst_v0, dst_v1, rows_a, rows_b,
             s_is0, s_is1, s_id0, s_id1, s_ga, s_gb):
    cid = lax.axis_index("c")
    sid = lax.axis_index("s")
    tid = cid * NS + sid
    base = sid * RPT
    out_base = cid * NP + base

    def fill_rows(buf, val16):
        def fill(i, carry):
            for j in range(D // 16):
                buf[i, pl.ds(j * 16, 16)] = val16
            return carry
        lax.fori_loop(0, CB, fill, 0)

    def zero_acc():
        off = 0
        for sz in _STRIPE_CHUNKS:
            pltpu.sync_copy(rows_a.at[pl.ds(0, sz)],
                            acc.at[pl.ds(base + off, sz)])
            off += sz

    def writeout(dest):
        off = 0
        for sz in _STRIPE_CHUNKS:
            pltpu.sync_copy(acc.at[pl.ds(base + off, sz)],
                            rows_a.at[pl.ds(0, sz)])
            pltpu.sync_copy(rows_a.at[pl.ds(0, sz)],
                            dest.at[pl.ds(out_base + off, sz)])
            off += sz

    def wait_idx(sem, buf):
        pltpu.make_async_copy(src_hbm.at[tid, 0], buf.at[0], sem).wait()

    def wait_gather(sem, src_idx, buf):
        pltpu.make_async_copy(x_hbm.at[src_idx.at[0]], buf, sem).wait()

    zeros16 = jnp.zeros((16,), jnp.float32)
    ones16 = jnp.ones((16,), jnp.float32)

    with jax.named_scope("sc_zero1"):
        fill_rows(rows_a, zeros16)
        zero_acc()
        plsc.subcore_barrier()

    pltpu.async_copy(src_hbm.at[tid, 0], src_v0.at[0], s_is0)
    pltpu.async_copy(dst_hbm.at[tid, 0], dst_v0.at[0], s_id0)
    pltpu.async_copy(src_hbm.at[tid, 1], src_v1.at[0], s_is1)
    pltpu.async_copy(dst_hbm.at[tid, 1], dst_v1.at[0], s_id1)
    wait_idx(s_is0, src_v0)
    pltpu.async_copy(x_hbm.at[src_v0.at[0]], rows_a, s_ga)

    def pbody(i, carry):
        g2 = 2 * i + 2
        g3 = 2 * i + 3
        wait_idx(s_is1, src_v1)
        wait_gather(s_ga, src_v0, rows_a)
        pltpu.async_copy(x_hbm.at[src_v1.at[0]], rows_b, s_gb)
        wait_idx(s_id0, dst_v0)
        pltpu.sync_copy(rows_a, acc.at[dst_v0.at[0]], add=True)
        pltpu.async_copy(src_hbm.at[tid, g2], src_v0.at[0], s_is0)
        pltpu.async_copy(dst_hbm.at[tid, g2], dst_v0.at[0], s_id0)
        wait_idx(s_is0, src_v0)
        wait_gather(s_gb, src_v1, rows_b)
        pltpu.async_copy(x_hbm.at[src_v0.at[0]], rows_a, s_ga)
        wait_idx(s_id1, dst_v1)
        pltpu.sync_copy(rows_b, acc.at[dst_v1.at[0]], add=True)
        pltpu.async_copy(src_hbm.at[tid, g3], src_v1.at[0], s_is1)
        pltpu.async_copy(dst_hbm.at[tid, g3], dst_v1.at[0], s_id1)
        return carry

    with jax.named_scope("sc_p1"):
        lax.fori_loop(0, NITER, pbody, 0)
        wait_gather(s_ga, src_v0, rows_a)
        wait_idx(s_id0, dst_v0)
        wait_idx(s_is1, src_v1)
        wait_idx(s_id1, dst_v1)
        plsc.subcore_barrier()
    with jax.named_scope("sc_wo1"):
        writeout(sum_out)

    with jax.named_scope("sc_zero2"):
        fill_rows(rows_a, zeros16)
        zero_acc()
        fill_rows(rows_b, ones16)
        plsc.subcore_barrier()

    pltpu.async_copy(dst_hbm.at[tid, 0], dst_v0.at[0], s_id0)
    pltpu.async_copy(dst_hbm.at[tid, 1], dst_v1.at[0], s_id1)

    def cbody(i, carry):
        g2 = 2 * i + 2
        g3 = 2 * i + 3
        wait_idx(s_id0, dst_v0)
        pltpu.sync_copy(rows_b, acc.at[dst_v0.at[0]], add=True)
        pltpu.async_copy(dst_hbm.at[tid, g2], dst_v0.at[0], s_id0)
        wait_idx(s_id1, dst_v1)
        pltpu.sync_copy(rows_b, acc.at[dst_v1.at[0]], add=True)
        pltpu.async_copy(dst_hbm.at[tid, g3], dst_v1.at[0], s_id1)
        return carry

    with jax.named_scope("sc_p2"):
        lax.fori_loop(0, NITER, cbody, 0)
        wait_idx(s_id0, dst_v0)
        wait_idx(s_id1, dst_v1)
        plsc.subcore_barrier()
    with jax.named_scope("sc_wo2"):
        writeout(cnt_out)


_sc_aggregate = functools.partial(
    pl.kernel,
    out_type=(jax.ShapeDtypeStruct((NC * NP, D), jnp.float32),
              jax.ShapeDtypeStruct((NC * NP, D), jnp.float32)),
    mesh=plsc.VectorSubcoreMesh(core_axis_name="c", subcore_axis_name="s",
                                num_cores=NC, num_subcores=NS),
    scratch_types=[
        pltpu.VMEM_SHARED((NP, D), jnp.float32),
        pltpu.VMEM((1, CB), jnp.int32),
        pltpu.VMEM((1, CB), jnp.int32),
        pltpu.VMEM((1, CB), jnp.int32),
        pltpu.VMEM((1, CB), jnp.int32),
        pltpu.VMEM((CB, D), jnp.float32),
        pltpu.VMEM((CB, D), jnp.float32),
        pltpu.SemaphoreType.DMA,
        pltpu.SemaphoreType.DMA,
        pltpu.SemaphoreType.DMA,
        pltpu.SemaphoreType.DMA,
        pltpu.SemaphoreType.DMA,
        pltpu.SemaphoreType.DMA,
    ],
)(_sc_body)


ROWS_BLK = 1000


def _tc_body(x_ref, s_ref, c_ref, w_ref, b_ref, o_ref):
    s = s_ref[0] + s_ref[1]
    c = c_ref[0, :, 0:1] + c_ref[1, :, 0:1]
    agg = s / jnp.maximum(c, 1.0)
    t = lax.dot_general(agg, w_ref[...], (((1,), (1,)), ((), ())),
                        preferred_element_type=jnp.float32)
    o_ref[...] = x_ref[...] + t + b_ref[...]


_tc_combine = pl.pallas_call(
    _tc_body,
    grid=(N_NODES // ROWS_BLK,),
    in_specs=[
        pl.BlockSpec((ROWS_BLK, D), lambda g: (g, 0)),
        pl.BlockSpec((NC, ROWS_BLK, D), lambda g: (0, g, 0)),
        pl.BlockSpec((NC, ROWS_BLK, D), lambda g: (0, g, 0)),
        pl.BlockSpec((D, D), lambda g: (0, 0)),
        pl.BlockSpec((1, D), lambda g: (0, 0)),
    ],
    out_specs=pl.BlockSpec((ROWS_BLK, D), lambda g: (g, 0)),
    out_shape=jax.ShapeDtypeStruct((N_NODES, D), jnp.float32),
)


def kernel(x, edge_index, W, b):
    src = edge_index[0].astype(jnp.int32)
    dst = edge_index[1].astype(jnp.int32)
    pad = NT * NCHUNK_SC * CB - N_EDGES
    src = jnp.concatenate([src, jnp.zeros((pad,), jnp.int32)])
    dst = jnp.concatenate([dst, jnp.full((pad,), N_NODES, jnp.int32)])
    src3 = jnp.concatenate(
        [src.reshape(NT, NCHUNK_SC, CB),
         jnp.zeros((NT, 2, CB), jnp.int32)], axis=1)
    dst3 = jnp.concatenate(
        [dst.reshape(NT, NCHUNK_SC, CB),
         jnp.full((NT, 2, CB), N_NODES, jnp.int32)], axis=1)
    sums, cnts = _sc_aggregate(x, src3, dst3)
    sums = sums.reshape(NC, NP, D)
    cnts = cnts.reshape(NC, NP, D)
    return _tc_combine(x, sums, cnts, W, b.reshape(1, D))
